# baseline (speedup 1.0000x reference)
import jax
import jax.numpy as jnp
from jax.experimental import pallas as pl
from jax.experimental.pallas import tpu as pltpu


def _dgi_kernel(adj_t_ref, adj_b_ref, f_ref, n_ref, w_ref, b2_ref, alpha_ref,
                dwt_ref, db_ref,
                sc1t_ref, sc1b_ref, sc2t_ref, sc2b_ref,
                s_ref, hpt_ref, hnt_ref, hpb_ref, hnb_ref, csum_ref):
    F = w_ref.shape[1]
    i = pl.program_id(0)
    nsteps = pl.num_programs(0)
    BM = adj_t_ref.shape[1]

    @pl.when(i == 0)
    def _():
        w = w_ref[:]
        s_ref[:, :F] = jnp.dot(
            f_ref[:], w, preferred_element_type=jnp.float32
        ).astype(jnp.bfloat16)
        s_ref[:, F:] = jnp.dot(
            n_ref[:], w, preferred_element_type=jnp.float32
        ).astype(jnp.bfloat16)
        csum_ref[:] = jnp.zeros_like(csum_ref)

    s = s_ref[:]
    b2 = b2_ref[:]
    alpha = alpha_ref[0, 0]
    row = i * BM

    def mm(a_ref, hp_all_ref, hn_all_ref):
        acc = jnp.dot(a_ref[0].astype(jnp.bfloat16), s,
                      preferred_element_type=jnp.float32)
        hblk = acc + b2
        hblk = jnp.where(hblk >= 0, hblk, alpha * hblk)
        hp = hblk[:, :F]
        hn = hblk[:, F:]
        hp_all_ref[pl.ds(row, BM), :] = hp.astype(jnp.bfloat16)
        hn_all_ref[pl.ds(row, BM), :] = hn.astype(jnp.bfloat16)
        return jnp.sum(hp, axis=0, keepdims=True)

    cs_t = mm(adj_t_ref, hpt_ref, hnt_ref)
    cs_b = mm(adj_b_ref, hpb_ref, hnb_ref)
    csum_ref[:] = csum_ref[:] + cs_t + cs_b

    @pl.when(i == nsteps - 1)
    def _():
        n_nodes = 2 * hpt_ref.shape[0]
        c = csum_ref[:] * (1.0 / n_nodes)
        sg = jax.nn.sigmoid(c)
        u = jnp.dot(sg, dwt_ref[:],
                    preferred_element_type=jnp.float32).T
        ub = u.astype(jnp.bfloat16)
        db = db_ref[0, 0]
        for h_ref, sc_ref in ((hpt_ref, sc1t_ref), (hpb_ref, sc1b_ref),
                              (hnt_ref, sc2t_ref), (hnb_ref, sc2b_ref)):
            sc_ref[:] = jnp.dot(h_ref[:], ub,
                                preferred_element_type=jnp.float32) + db


def kernel(features, negative_features, adj, W_gcn, b_gcn, prelu_alpha, disc_W, disc_b):
    B, N, IN_F = features.shape
    OUT_F = W_gcn.shape[1]
    H = N // 2
    f2 = features.reshape(N, IN_F)
    n2 = negative_features.reshape(N, IN_F)
    adj3 = adj.reshape(2, H, N)
    b2 = jnp.concatenate([b_gcn, b_gcn]).reshape(1, 2 * OUT_F)
    alpha = prelu_alpha.reshape(1, 1)
    db = disc_b.reshape(1, 1)
    dwt = disc_W.T

    BM = 200
    const = lambda shape: pl.BlockSpec(shape, lambda i: tuple(0 for _ in shape))
    sc1t, sc1b, sc2t, sc2b = pl.pallas_call(
        _dgi_kernel,
        grid=(H // BM,),
        in_specs=[
            pl.BlockSpec((1, BM, N), lambda i: (0, i, 0)),
            pl.BlockSpec((1, BM, N), lambda i: (1, i, 0)),
            const((N, IN_F)),
            const((N, IN_F)),
            const((IN_F, OUT_F)),
            const((1, 2 * OUT_F)),
            const((1, 1)),
            const((OUT_F, OUT_F)),
            const((1, 1)),
        ],
        out_specs=[const((H, 1))] * 4,
        out_shape=[jax.ShapeDtypeStruct((H, 1), jnp.float32)] * 4,
        scratch_shapes=[
            pltpu.VMEM((N, 2 * OUT_F), jnp.bfloat16),
            pltpu.VMEM((H, OUT_F), jnp.bfloat16),
            pltpu.VMEM((H, OUT_F), jnp.bfloat16),
            pltpu.VMEM((H, OUT_F), jnp.bfloat16),
            pltpu.VMEM((H, OUT_F), jnp.bfloat16),
            pltpu.VMEM((1, OUT_F), jnp.float32),
        ],
        compiler_params=pltpu.CompilerParams(
            vmem_limit_bytes=100 * 1024 * 1024),
    )(adj3, adj3, f2, n2, W_gcn, b2, alpha, dwt, db)

    return jnp.concatenate(
        [sc1t.reshape(1, H), sc1b.reshape(1, H),
         sc2t.reshape(1, H), sc2b.reshape(1, H)], axis=1)
